# baseline (speedup 1.0000x reference)
import functools

import jax
import jax.numpy as jnp
from jax import lax
from jax.experimental import pallas as pl
from jax.experimental.pallas import tpu as pltpu
from jax.experimental.pallas import tpu_sc as plsc

H = 256


@functools.lru_cache(maxsize=None)
def _make_sc_gather(V, N, CH):
    info = plsc.get_sparse_core_info()
    NC, NS = info.num_cores, info.num_subcores
    NW = NC * NS
    per_w = N // NW
    n_ch = per_w // CH
    assert per_w % CH == 0 and N % NW == 0 and CH <= 128
    mesh = plsc.VectorSubcoreMesh(core_axis_name="c", subcore_axis_name="s")

    @functools.partial(
        pl.kernel,
        mesh=mesh,
        out_type=jax.ShapeDtypeStruct((N, H), jnp.float32),
        scratch_types=[
            pltpu.VMEM((n_ch, CH), jnp.int32),
            pltpu.VMEM((3, CH, H), jnp.float32),
            pltpu.SemaphoreType.DMA,
            pltpu.SemaphoreType.DMA,
            pltpu.SemaphoreType.DMA,
            pltpu.SemaphoreType.DMA,
            pltpu.SemaphoreType.DMA,
            pltpu.SemaphoreType.DMA,
        ],
    )
    def gather_k(table_hbm, idx_hbm, out_hbm, idx_v, rows_v,
                 g0, g1, g2, o0, o1, o2):
        wid = lax.axis_index("s") * NC + lax.axis_index("c")
        base = wid * per_w
        gsems = (g0, g1, g2)
        osems = (o0, o1, o2)

        pltpu.sync_copy(idx_hbm.at[wid], idx_v)
        gathers = [None, None, None]
        wbs = [None, None, None]
        for k0 in range(min(2, n_ch)):
            gathers[k0] = pltpu.async_copy(
                table_hbm.at[idx_v.at[k0]], rows_v.at[k0], gsems[k0]
            )
        for k in range(n_ch):
            b = k % 3
            if k + 2 < n_ch:
                nb = (k + 2) % 3
                if wbs[nb] is not None:
                    wbs[nb].wait()
                gathers[nb] = pltpu.async_copy(
                    table_hbm.at[idx_v.at[k + 2]], rows_v.at[nb], gsems[nb]
                )
            gathers[b].wait()
            wbs[b] = pltpu.async_copy(
                rows_v.at[b], out_hbm.at[pl.ds(base + k * CH, CH)], osems[b]
            )
        for w in wbs:
            if w is not None:
                w.wait()

    return gather_k


def _dot(a, b):
    return jnp.dot(a.astype(jnp.bfloat16), b, preferred_element_type=jnp.float32)


def _sigmoid(x):
    return 0.5 * jnp.tanh(0.5 * x) + 0.5


def _lstm_cell(iou, ct):
    i, o, u = iou[:, :H], iou[:, H:2 * H], iou[:, 2 * H:]
    c = _sigmoid(i) * jnp.tanh(u) + ct
    h = _sigmoid(o) * jnp.tanh(c)
    return h, c


def _levels(h, c, w_ref, b_ref, n_lev):
    for _ in range(n_lev):
        k = h.shape[0] // 2
        h2 = h.reshape(k, 2 * H)
        c2 = c.reshape(k, 2 * H)
        g = _dot(h2, w_ref[...]) + b_ref[...]
        f = _sigmoid(g[:, :2 * H])
        fc = f * c2
        ct = fc[:, :H] + fc[:, H:]
        h, c = _lstm_cell(g[:, 2 * H:], ct)
    return h, c


def _blockA_body(x_ref, wl_ref, w_ref, b_ref, h_ref, c_ref):
    iou = _dot(x_ref[...], wl_ref[...]) + b_ref[...][:, 2 * H:]
    h, c = _lstm_cell(iou, 0.0)
    h, c = _levels(h, c, w_ref, b_ref, 4)
    h_ref[...] = h
    c_ref[...] = c


def _make_blockA2_body(NP):
    def body(x_ref, wl_ref, w_ref, b_ref, *rest):
        h6refs = rest[:NP]
        c6refs = rest[NP:2 * NP]
        wo_ref, bo_ref, out_ref, h6s, c6s = rest[2 * NP:]
        i = pl.program_id(0)
        n = pl.num_programs(0)
        iou = _dot(x_ref[...], wl_ref[...]) + b_ref[...][:, 2 * H:]
        h, c = _lstm_cell(iou, 0.0)
        h, c = _levels(h, c, w_ref, b_ref, 4)
        ob = h.shape[0]
        h6s[pl.ds(i * ob, ob), :] = h
        c6s[pl.ds(i * ob, ob), :] = c

        @pl.when(i == n - 1)
        def _tail():
            hh = jnp.concatenate([r[...] for r in h6refs] + [h6s[...]], axis=0)
            cc = jnp.concatenate([r[...] for r in c6refs] + [c6s[...]], axis=0)
            hr, _ = _levels(hh, cc, w_ref, b_ref, 6)
            logits = (jnp.dot(hr, wo_ref[...], preferred_element_type=jnp.float32)
                      + bo_ref[...])
            mx = jnp.max(logits, axis=-1, keepdims=True)
            e = jnp.exp(logits - mx)
            out_ref[...] = logits - mx - jnp.log(jnp.sum(e, axis=-1, keepdims=True))

    return body


def kernel(label, depth, batch, emb, W_iou, U_iou, b_iou, U_f_w, U_f_b, out_w, out_b):
    D = 10
    M = 2 ** (D + 1) - 1
    B = label.shape[0] // M
    NL = B * 2 ** D
    out_size = out_w.shape[0]

    leaf_labels = label.reshape(B, M)[:, M // 2:].reshape(-1).astype(jnp.int32)

    CH = 128
    info = plsc.get_sparse_core_info()
    NW = info.num_cores * info.num_subcores
    SPLITS = (12, 52)
    NG = len(SPLITS)
    xs = []
    off = 0
    for bt in SPLITS:
        ng = bt * 2 ** D
        xs.append(_make_sc_gather(emb.shape[0], ng, CH)(
            emb, leaf_labels[off:off + ng].reshape(NW, ng // (NW * CH), CH)))
        off += ng

    WlT = W_iou.T.astype(jnp.bfloat16)
    Wall = jnp.concatenate([U_f_w, U_iou], axis=0).T.astype(jnp.bfloat16)
    ball = jnp.concatenate([U_f_b.reshape(1, 2 * H), b_iou.reshape(1, 3 * H)],
                           axis=1)

    T = 4
    LB = T * 2 ** D
    OB = T * 2 ** 6
    parts = [pl.pallas_call(
        _blockA_body,
        grid=(bt // T,),
        in_specs=[
            pl.BlockSpec((LB, H), lambda i: (i, 0)),
            pl.BlockSpec((H, 3 * H), lambda i: (0, 0)),
            pl.BlockSpec((2 * H, 5 * H), lambda i: (0, 0)),
            pl.BlockSpec((1, 5 * H), lambda i: (0, 0)),
        ],
        out_specs=[
            pl.BlockSpec((OB, H), lambda i: (i, 0)),
            pl.BlockSpec((OB, H), lambda i: (i, 0)),
        ],
        out_shape=[jax.ShapeDtypeStruct((bt * 2 ** 6, H), jnp.float32)] * 2,
    )(xg, WlT, Wall, ball) for bt, xg in zip(SPLITS[:NG - 1], xs[:NG - 1])]

    WoPad = jnp.zeros((H, 128), jnp.float32).at[:, :out_size].set(out_w.T)
    boPad = jnp.full((1, 128), -1e30, jnp.float32).at[0, :out_size].set(out_b)
    NP = NG - 1
    BLAST = SPLITS[-1]
    NSC = BLAST * 2 ** 6
    ls = pl.pallas_call(
        _make_blockA2_body(NP),
        grid=(BLAST // T,),
        in_specs=[
            pl.BlockSpec((LB, H), lambda i: (i, 0)),
            pl.BlockSpec((H, 3 * H), lambda i: (0, 0)),
            pl.BlockSpec((2 * H, 5 * H), lambda i: (0, 0)),
            pl.BlockSpec((1, 5 * H), lambda i: (0, 0)),
        ] + [pl.BlockSpec((bt * 2 ** 6, H), lambda i: (0, 0))
             for bt in list(SPLITS[:NG - 1]) * 2] + [
            pl.BlockSpec((H, 128), lambda i: (0, 0)),
            pl.BlockSpec((1, 128), lambda i: (0, 0)),
        ],
        out_specs=pl.BlockSpec((B, 128), lambda i: (0, 0)),
        out_shape=jax.ShapeDtypeStruct((B, 128), jnp.float32),
        scratch_shapes=[
            pltpu.VMEM((NSC, H), jnp.float32),
            pltpu.VMEM((NSC, H), jnp.float32),
        ],
    )(xs[NG - 1], WlT, Wall, ball,
      *[p[0] for p in parts], *[p[1] for p in parts], WoPad, boPad)
    return ls[:, :out_size]


# speedup vs baseline: 1.0480x; 1.0480x over previous
import functools

import jax
import jax.numpy as jnp
from jax import lax
from jax.experimental import pallas as pl
from jax.experimental.pallas import tpu as pltpu
from jax.experimental.pallas import tpu_sc as plsc

H = 256


@functools.lru_cache(maxsize=None)
def _make_sc_gather(V, N, CH):
    info = plsc.get_sparse_core_info()
    NC, NS = info.num_cores, info.num_subcores
    NW = NC * NS
    per_w = N // NW
    n_ch = per_w // CH
    assert per_w % CH == 0 and N % NW == 0 and CH <= 128
    mesh = plsc.VectorSubcoreMesh(core_axis_name="c", subcore_axis_name="s")

    @functools.partial(
        pl.kernel,
        mesh=mesh,
        out_type=jax.ShapeDtypeStruct((N, H), jnp.float32),
        scratch_types=[
            pltpu.VMEM((n_ch, CH), jnp.int32),
            pltpu.VMEM((3, CH, H), jnp.float32),
            pltpu.SemaphoreType.DMA,
            pltpu.SemaphoreType.DMA,
            pltpu.SemaphoreType.DMA,
            pltpu.SemaphoreType.DMA,
            pltpu.SemaphoreType.DMA,
            pltpu.SemaphoreType.DMA,
        ],
    )
    def gather_k(table_hbm, idx_hbm, out_hbm, idx_v, rows_v,
                 g0, g1, g2, o0, o1, o2):
        wid = lax.axis_index("s") * NC + lax.axis_index("c")
        base = wid * per_w
        gsems = (g0, g1, g2)
        osems = (o0, o1, o2)

        pltpu.sync_copy(idx_hbm.at[wid], idx_v)
        gathers = [None, None, None]
        wbs = [None, None, None]
        for k0 in range(min(2, n_ch)):
            gathers[k0] = pltpu.async_copy(
                table_hbm.at[idx_v.at[k0]], rows_v.at[k0], gsems[k0]
            )
        for k in range(n_ch):
            b = k % 3
            if k + 2 < n_ch:
                nb = (k + 2) % 3
                if wbs[nb] is not None:
                    wbs[nb].wait()
                gathers[nb] = pltpu.async_copy(
                    table_hbm.at[idx_v.at[k + 2]], rows_v.at[nb], gsems[nb]
                )
            gathers[b].wait()
            wbs[b] = pltpu.async_copy(
                rows_v.at[b], out_hbm.at[pl.ds(base + k * CH, CH)], osems[b]
            )
        for w in wbs:
            if w is not None:
                w.wait()

    return gather_k


def _dot(a, b):
    return jnp.dot(a.astype(jnp.bfloat16), b, preferred_element_type=jnp.float32)


def _sigmoid(x):
    return 0.5 * jnp.tanh(0.5 * x) + 0.5


def _lstm_cell(iou, ct):
    i, o, u = iou[:, :H], iou[:, H:2 * H], iou[:, 2 * H:]
    c = _sigmoid(i) * jnp.tanh(u) + ct
    h = _sigmoid(o) * jnp.tanh(c)
    return h, c


def _levels(h, c, w_ref, b_ref, n_lev):
    for _ in range(n_lev):
        k = h.shape[0] // 2
        h2 = h.reshape(k, 2 * H)
        c2 = c.reshape(k, 2 * H)
        g = _dot(h2, w_ref[...]) + b_ref[...]
        f = _sigmoid(g[:, :2 * H])
        fc = f * c2
        ct = fc[:, :H] + fc[:, H:]
        h, c = _lstm_cell(g[:, 2 * H:], ct)
    return h, c


def _blockA_body(x_ref, wl_ref, w_ref, b_ref, h_ref, c_ref):
    iou = _dot(x_ref[...], wl_ref[...]) + b_ref[...][:, 2 * H:]
    h, c = _lstm_cell(iou, 0.0)
    h, c = _levels(h, c, w_ref, b_ref, 4)
    h_ref[...] = h
    c_ref[...] = c


def _make_blockA2_body(NP):
    def body(x_ref, wl_ref, w_ref, b_ref, *rest):
        h6refs = rest[:NP]
        c6refs = rest[NP:2 * NP]
        wo_ref, bo_ref, out_ref, h6s, c6s = rest[2 * NP:]
        i = pl.program_id(0)
        n = pl.num_programs(0)
        iou = _dot(x_ref[...], wl_ref[...]) + b_ref[...][:, 2 * H:]
        h, c = _lstm_cell(iou, 0.0)
        h, c = _levels(h, c, w_ref, b_ref, 4)
        ob = h.shape[0]
        h6s[pl.ds(i * ob, ob), :] = h
        c6s[pl.ds(i * ob, ob), :] = c

        @pl.when(i == n - 1)
        def _tail():
            hh = jnp.concatenate([r[...] for r in h6refs] + [h6s[...]], axis=0)
            cc = jnp.concatenate([r[...] for r in c6refs] + [c6s[...]], axis=0)
            hr, _ = _levels(hh, cc, w_ref, b_ref, 6)
            logits = (jnp.dot(hr, wo_ref[...], preferred_element_type=jnp.float32)
                      + bo_ref[...])
            mx = jnp.max(logits, axis=-1, keepdims=True)
            e = jnp.exp(logits - mx)
            out_ref[...] = logits - mx - jnp.log(jnp.sum(e, axis=-1, keepdims=True))

    return body


def kernel(label, depth, batch, emb, W_iou, U_iou, b_iou, U_f_w, U_f_b, out_w, out_b):
    D = 10
    M = 2 ** (D + 1) - 1
    B = label.shape[0] // M
    NL = B * 2 ** D
    out_size = out_w.shape[0]

    leaf_labels = label.reshape(B, M)[:, M // 2:].reshape(-1).astype(jnp.int32)

    CH = 128
    info = plsc.get_sparse_core_info()
    NW = info.num_cores * info.num_subcores
    SPLITS = (16, 48)
    NG = len(SPLITS)
    xs = []
    off = 0
    for bt in SPLITS:
        ng = bt * 2 ** D
        xs.append(_make_sc_gather(emb.shape[0], ng, CH)(
            emb, leaf_labels[off:off + ng].reshape(NW, ng // (NW * CH), CH)))
        off += ng

    WlT = W_iou.T.astype(jnp.bfloat16)
    Wall = jnp.concatenate([U_f_w, U_iou], axis=0).T.astype(jnp.bfloat16)
    ball = jnp.concatenate([U_f_b.reshape(1, 2 * H), b_iou.reshape(1, 3 * H)],
                           axis=1)

    T = 4
    LB = T * 2 ** D
    OB = T * 2 ** 6
    parts = [pl.pallas_call(
        _blockA_body,
        grid=(bt // T,),
        in_specs=[
            pl.BlockSpec((LB, H), lambda i: (i, 0)),
            pl.BlockSpec((H, 3 * H), lambda i: (0, 0)),
            pl.BlockSpec((2 * H, 5 * H), lambda i: (0, 0)),
            pl.BlockSpec((1, 5 * H), lambda i: (0, 0)),
        ],
        out_specs=[
            pl.BlockSpec((OB, H), lambda i: (i, 0)),
            pl.BlockSpec((OB, H), lambda i: (i, 0)),
        ],
        out_shape=[jax.ShapeDtypeStruct((bt * 2 ** 6, H), jnp.float32)] * 2,
    )(xg, WlT, Wall, ball) for bt, xg in zip(SPLITS[:NG - 1], xs[:NG - 1])]

    WoPad = jnp.zeros((H, 128), jnp.float32).at[:, :out_size].set(out_w.T)
    boPad = jnp.full((1, 128), -1e30, jnp.float32).at[0, :out_size].set(out_b)
    NP = NG - 1
    BLAST = SPLITS[-1]
    NSC = BLAST * 2 ** 6
    ls = pl.pallas_call(
        _make_blockA2_body(NP),
        grid=(BLAST // T,),
        in_specs=[
            pl.BlockSpec((LB, H), lambda i: (i, 0)),
            pl.BlockSpec((H, 3 * H), lambda i: (0, 0)),
            pl.BlockSpec((2 * H, 5 * H), lambda i: (0, 0)),
            pl.BlockSpec((1, 5 * H), lambda i: (0, 0)),
        ] + [pl.BlockSpec((bt * 2 ** 6, H), lambda i: (0, 0))
             for bt in list(SPLITS[:NG - 1]) * 2] + [
            pl.BlockSpec((H, 128), lambda i: (0, 0)),
            pl.BlockSpec((1, 128), lambda i: (0, 0)),
        ],
        out_specs=pl.BlockSpec((B, 128), lambda i: (0, 0)),
        out_shape=jax.ShapeDtypeStruct((B, 128), jnp.float32),
        scratch_shapes=[
            pltpu.VMEM((NSC, H), jnp.float32),
            pltpu.VMEM((NSC, H), jnp.float32),
        ],
    )(xs[NG - 1], WlT, Wall, ball,
      *[p[0] for p in parts], *[p[1] for p in parts], WoPad, boPad)
    return ls[:, :out_size]
